# baseline (speedup 1.0000x reference)
import functools

import jax
import jax.numpy as jnp
from jax import lax
from jax.experimental import pallas as pl
from jax.experimental.pallas import tpu as pltpu
from jax.experimental.pallas import tpu_sc as plsc

_NC, _NS = 2, 16
_R = 8


def kernel(embeddings, pos_table):
    B, S, D = embeddings.shape
    assert B == 4
    nw = _NC * _NS
    s_per_w = S // nw
    n_chunks = s_per_w // _R
    assert n_chunks % 2 == 0
    lanes_per_row = D // 16

    mesh = plsc.VectorSubcoreMesh(
        core_axis_name="c", subcore_axis_name="s", num_cores=_NC, num_subcores=_NS
    )

    @functools.partial(
        pl.kernel,
        out_type=jax.ShapeDtypeStruct((B, S, D), jnp.float32),
        mesh=mesh,
        scratch_types=[
            [pltpu.VMEM((_R, D), jnp.float32) for _ in range(2)],
            [[pltpu.VMEM((_R, D), jnp.float32) for _ in range(4)] for _ in range(2)],
            [pltpu.SemaphoreType.DMA for _ in range(2)],
            [[pltpu.SemaphoreType.DMA for _ in range(4)] for _ in range(2)],
            [[pltpu.SemaphoreType.DMA for _ in range(4)] for _ in range(2)],
        ],
    )
    def sc_add(emb_hbm, pos_hbm, out_hbm, pbufs, ebufs, psems, isems, osems):
        wid = lax.axis_index("s") * _NC + lax.axis_index("c")
        s_base = wid * s_per_w

        def start_chunk_in(i, bank):
            s0 = s_base + i * _R
            pltpu.async_copy(pos_hbm.at[pl.ds(s0, _R)], pbufs[bank], psems[bank])
            for k in range(4):
                pltpu.async_copy(
                    emb_hbm.at[k, pl.ds(s0, _R)], ebufs[bank][k], isems[bank][k]
                )

        start_chunk_in(0, 0)

        def pair(i2, carry):
            for bank in range(2):
                i = 2 * i2 + bank
                s0 = s_base + i * _R
                other = 1 - bank

                pltpu.make_async_copy(
                    pos_hbm.at[pl.ds(s0, _R)], pbufs[bank], psems[bank]
                ).wait()
                for k in range(4):
                    pltpu.make_async_copy(
                        emb_hbm.at[k, pl.ds(s0, _R)], ebufs[bank][k], isems[bank][k]
                    ).wait()

                    def add_row(r, carry2, bank=bank, k=k):
                        for j in range(lanes_per_row):
                            plsc.addupdate(
                                ebufs[bank][k].at[r, pl.ds(j * 16, 16)],
                                pbufs[bank][r, pl.ds(j * 16, 16)],
                            )
                        return carry2

                    lax.fori_loop(0, _R, add_row, 0)
                    pltpu.async_copy(
                        ebufs[bank][k], out_hbm.at[k, pl.ds(s0, _R)], osems[bank][k]
                    )

                    @pl.when(i + 1 < n_chunks)
                    def _(i=i, bank=bank, other=other, k=k):
                        s_prev = s_base + (i - 1) * _R
                        s_next = s_base + (i + 1) * _R

                        @pl.when(i >= 1)
                        def _():
                            pltpu.make_async_copy(
                                ebufs[other][k],
                                out_hbm.at[k, pl.ds(s_prev, _R)],
                                osems[other][k],
                            ).wait()

                        pltpu.async_copy(
                            emb_hbm.at[k, pl.ds(s_next, _R)],
                            ebufs[other][k],
                            isems[other][k],
                        )
                        if k == 0:
                            pltpu.async_copy(
                                pos_hbm.at[pl.ds(s_next, _R)],
                                pbufs[other],
                                psems[other],
                            )
            return carry

        lax.fori_loop(0, n_chunks // 2, pair, 0)

        for bank in range(2):
            i_last = n_chunks - 2 + bank
            s_last = s_base + i_last * _R
            for k in range(4):
                pltpu.make_async_copy(
                    ebufs[bank][k],
                    out_hbm.at[k, pl.ds(s_last, _R)],
                    osems[bank][k],
                ).wait()

    return sc_add(embeddings, pos_table)
